# Initial kernel scaffold; baseline (speedup 1.0000x reference)
import jax
import jax.numpy as jnp
from jax.experimental import pallas as pl


def kernel(x, pos_edge_index, W1, b1, W2, b2):
    raise NotImplementedError("write your pallas kernel here")


# speedup vs baseline: 5.4398x; 5.4398x over previous
import functools

import jax
import jax.numpy as jnp
from jax import lax
from jax.experimental import pallas as pl
from jax.experimental.pallas import tpu as pltpu
from jax.experimental.pallas import tpu_sc as plsc

NW = 32
NTILE = 16


def _np_pad(n):
  return (n // 512 + 1) * 512


def _make_deg(n_pad, e_pad, k):
  ew = e_pad // NW
  nch = ew // k
  rows = n_pad // NTILE
  mesh = plsc.VectorSubcoreMesh(core_axis_name="c", subcore_axis_name="s")

  @functools.partial(
      pl.kernel, mesh=mesh,
      out_type=[
          jax.ShapeDtypeStruct((n_pad,), jnp.float32),
          jax.ShapeDtypeStruct((n_pad,), jnp.float32),
      ],
      scratch_types=[
          pltpu.VMEM((k,), jnp.int32),
          pltpu.VMEM((k,), jnp.float32),
          pltpu.VMEM((n_pad // NTILE,), jnp.float32),
          pltpu.VMEM_SHARED((n_pad,), jnp.float32),
      ],
  )
  def deg_kernel(dst_hbm, ones_hbm, zeros_hbm, out0_hbm, out1_hbm,
                 dstv, onesv, zbuf, acc):
    cid = lax.axis_index("c")
    sid = lax.axis_index("s")
    wid = sid * 2 + cid
    base_r = sid * rows
    pltpu.sync_copy(zeros_hbm, zbuf)
    pltpu.sync_copy(zbuf, acc.at[pl.ds(base_r, rows)])
    pltpu.sync_copy(ones_hbm, onesv)
    plsc.subcore_barrier()
    ebase = wid * ew

    def body(c, carry):
      off = pl.multiple_of(ebase + c * k, 8)
      pltpu.sync_copy(dst_hbm.at[pl.ds(off, k)], dstv)
      pltpu.sync_copy(onesv, acc.at[dstv], add=True)
      return carry

    lax.fori_loop(0, nch, body, 0)
    plsc.subcore_barrier()
    pltpu.sync_copy(acc.at[pl.ds(base_r, rows)], zbuf)

    @pl.when(cid == 0)
    def _():
      pltpu.sync_copy(zbuf, out0_hbm.at[pl.ds(base_r, rows)])

    @pl.when(cid == 1)
    def _():
      pltpu.sync_copy(zbuf, out1_hbm.at[pl.ds(base_r, rows)])

  return deg_kernel


def _make_scatter(n_pad, e_pad, d, k):
  ew = e_pad // NTILE
  nch = ew // k
  hn = n_pad // 2
  rows = hn // NTILE
  zr = rows // 2
  acc_rows = hn + 8
  mesh = plsc.VectorSubcoreMesh(core_axis_name="c", subcore_axis_name="s")

  @functools.partial(
      pl.kernel, mesh=mesh,
      out_type=jax.ShapeDtypeStruct((n_pad, d), jnp.float32),
      scratch_types=[
          pltpu.VMEM((k,), jnp.int32),
          pltpu.VMEM((k,), jnp.int32),
          pltpu.VMEM((k,), jnp.int32),
          pltpu.VMEM((k, d), jnp.float32),
          pltpu.VMEM((zr, d), jnp.float32),
          pltpu.VMEM_SHARED((acc_rows, d), jnp.float32),
          pltpu.SemaphoreType.DMA,
      ],
  )
  def scat_kernel(hs_hbm, src_hbm, dst_hbm, zeros_hbm, out_hbm,
                  srcv, dstv, idxv, rowsv, zbuf, acc, sem):
    cid = lax.axis_index("c")
    sid = lax.axis_index("s")
    nbase = cid * hn
    base_r = sid * rows
    pltpu.sync_copy(zeros_hbm, zbuf)
    for z in range(rows // zr):
      pltpu.sync_copy(zbuf, acc.at[pl.ds(base_r + z * zr, zr)])
    @pl.when(sid == 0)
    def _():
      pltpu.sync_copy(zbuf.at[pl.ds(0, 8)], acc.at[pl.ds(hn, 8)])
    plsc.subcore_barrier()
    ebase = sid * ew

    def body(c, carry):
      off = pl.multiple_of(ebase + c * k, 8)
      pltpu.sync_copy(src_hbm.at[pl.ds(off, k)], srcv)
      pltpu.sync_copy(dst_hbm.at[pl.ds(off, k)], dstv)

      def remap(j, carry2):
        dv = dstv[pl.ds(j * 16, 16)] - nbase
        ok = (dv >= 0) & (dv < hn)
        idxv[pl.ds(j * 16, 16)] = jnp.where(ok, dv, hn)
        return carry2

      lax.fori_loop(0, k // 16, remap, 0)
      pltpu.async_copy(hs_hbm.at[srcv], rowsv, sem).wait()
      pltpu.sync_copy(rowsv, acc.at[idxv], add=True)
      return carry

    lax.fori_loop(0, nch, body, 0)
    plsc.subcore_barrier()
    for z in range(rows // zr):
      pltpu.sync_copy(acc.at[pl.ds(base_r + z * zr, zr)], zbuf)
      pltpu.sync_copy(zbuf, out_hbm.at[pl.ds(nbase + base_r + z * zr, zr)])

  return scat_kernel


def _tc_a(xp, d0, d1):
  n_pad, df = xp.shape

  def body(x_ref, d0_ref, d1_ref, xs_ref, dinv_ref):
    deg = d0_ref[...] + d1_ref[...] + 1.0
    dinv = lax.rsqrt(deg)
    xs_ref[...] = dinv * x_ref[...]
    dinv_ref[...] = dinv

  return pl.pallas_call(
      body,
      out_shape=[
          jax.ShapeDtypeStruct((n_pad, df), jnp.float32),
          jax.ShapeDtypeStruct((n_pad, 1), jnp.float32),
      ],
  )(xp, d0, d1)


def _tc_b(s1, xp, dinv, w1, b1, w2):
  n_pad, df = xp.shape

  def body(a_ref, x_ref, d_ref, w1_ref, b1_ref, w2_ref, hs_ref, self_ref):
    dv = d_ref[...]
    t = dv * a_ref[...] + (dv * dv) * x_ref[...]
    z1 = jnp.maximum(
        jnp.dot(t, w1_ref[...], preferred_element_type=jnp.float32)
        + b1_ref[...], 0.0)
    h2 = jnp.dot(z1, w2_ref[...], preferred_element_type=jnp.float32)
    hs_ref[...] = dv * h2
    self_ref[...] = (dv * dv) * h2

  return pl.pallas_call(
      body,
      out_shape=[
          jax.ShapeDtypeStruct((n_pad, df), jnp.float32),
          jax.ShapeDtypeStruct((n_pad, df), jnp.float32),
      ],
  )(s1, xp, dinv, w1, b1, w2)


def _tc_c(s2, self2, dinv, b2):
  n_pad, df = s2.shape

  def body(a_ref, s_ref, d_ref, bias_ref, out_ref):
    out_ref[...] = d_ref[...] * a_ref[...] + s_ref[...] + bias_ref[...]

  return pl.pallas_call(
      body,
      out_shape=jax.ShapeDtypeStruct((n_pad, df), jnp.float32),
  )(s2, self2, dinv, b2)


def kernel(x, pos_edge_index, W1, b1, W2, b2):
  n, df = x.shape
  dh = W1.shape[1]
  e = pos_edge_index.shape[1]
  n_pad = _np_pad(n)

  k_deg = 2048
  k128 = 512
  kmax = max(k_deg, k128)
  e_pad = -(-e // (NW * kmax)) * (NW * kmax)

  src = pos_edge_index[0]
  dst = pos_edge_index[1]
  if e_pad != e:
    pad = jnp.full((e_pad - e,), n, dtype=jnp.int32)
    src = jnp.concatenate([src, pad])
    dst = jnp.concatenate([dst, pad])

  xp = jnp.pad(x, ((0, n_pad - n), (0, 0)))
  rows = n_pad // NTILE
  z1d = jnp.zeros((rows,), jnp.float32)
  z128 = jnp.zeros((n_pad // 2 // NTILE // 2, df), jnp.float32)
  ones = jnp.ones((k_deg,), jnp.float32)

  degp0, degp1 = _make_deg(n_pad, e_pad, k_deg)(dst, ones, z1d)
  d0 = degp0[:, None]
  d1 = degp1[:, None]

  xs, dinv = _tc_a(xp, d0, d1)
  scat = _make_scatter(n_pad, e_pad, df, k128)
  s1 = scat(xs, src, dst, z128)
  hs2, self2 = _tc_b(s1, xp, dinv, W1, b1.reshape(1, dh), W2)
  s2 = scat(hs2, src, dst, z128)
  out = _tc_c(s2, self2, dinv, b2.reshape(1, df))
  return out[:n]


# speedup vs baseline: 5.8293x; 1.0716x over previous
import functools

import jax
import jax.numpy as jnp
from jax import lax
from jax.experimental import pallas as pl
from jax.experimental.pallas import tpu as pltpu
from jax.experimental.pallas import tpu_sc as plsc

NW = 32
NTILE = 16


def _np_pad(n):
  return (n // 512 + 1) * 512


def _make_deg(n_pad, e_pad, k):
  ew = e_pad // NW
  nch = ew // k
  rows = n_pad // NTILE
  mesh = plsc.VectorSubcoreMesh(core_axis_name="c", subcore_axis_name="s")

  @functools.partial(
      pl.kernel, mesh=mesh,
      out_type=[
          jax.ShapeDtypeStruct((n_pad,), jnp.float32),
          jax.ShapeDtypeStruct((n_pad,), jnp.float32),
      ],
      scratch_types=[
          pltpu.VMEM((k,), jnp.int32),
          pltpu.VMEM((k,), jnp.float32),
          pltpu.VMEM((n_pad // NTILE,), jnp.float32),
          pltpu.VMEM_SHARED((n_pad,), jnp.float32),
      ],
  )
  def deg_kernel(dst_hbm, ones_hbm, zeros_hbm, out0_hbm, out1_hbm,
                 dstv, onesv, zbuf, acc):
    cid = lax.axis_index("c")
    sid = lax.axis_index("s")
    wid = sid * 2 + cid
    base_r = sid * rows
    pltpu.sync_copy(zeros_hbm, zbuf)
    pltpu.sync_copy(zbuf, acc.at[pl.ds(base_r, rows)])
    pltpu.sync_copy(ones_hbm, onesv)
    plsc.subcore_barrier()
    ebase = wid * ew

    def body(c, carry):
      off = pl.multiple_of(ebase + c * k, 8)
      pltpu.sync_copy(dst_hbm.at[pl.ds(off, k)], dstv)
      pltpu.sync_copy(onesv, acc.at[dstv], add=True)
      return carry

    lax.fori_loop(0, nch, body, 0)
    plsc.subcore_barrier()
    pltpu.sync_copy(acc.at[pl.ds(base_r, rows)], zbuf)

    @pl.when(cid == 0)
    def _():
      pltpu.sync_copy(zbuf, out0_hbm.at[pl.ds(base_r, rows)])

    @pl.when(cid == 1)
    def _():
      pltpu.sync_copy(zbuf, out1_hbm.at[pl.ds(base_r, rows)])

  return deg_kernel


def _make_scatter(n_pad, e_pad, d, k):
  ew = e_pad // NTILE
  nch = ew // k
  assert nch % 2 == 0
  hn = n_pad // 2
  rows = hn // NTILE
  zr = rows // 2
  acc_rows = hn + 8
  mesh = plsc.VectorSubcoreMesh(core_axis_name="c", subcore_axis_name="s")

  @functools.partial(
      pl.kernel, mesh=mesh,
      out_type=jax.ShapeDtypeStruct((n_pad, d), jnp.float32),
      scratch_types=[
          pltpu.VMEM((k,), jnp.int32),
          pltpu.VMEM((k,), jnp.int32),
          pltpu.VMEM((k,), jnp.int32),
          pltpu.VMEM((k,), jnp.int32),
          pltpu.VMEM((k,), jnp.int32),
          pltpu.VMEM((k,), jnp.int32),
          pltpu.VMEM((k, d), jnp.float32),
          pltpu.VMEM((k, d), jnp.float32),
          pltpu.VMEM((zr, d), jnp.float32),
          pltpu.VMEM_SHARED((acc_rows, d), jnp.float32),
          pltpu.SemaphoreType.DMA,
          pltpu.SemaphoreType.DMA,
          pltpu.SemaphoreType.DMA,
          pltpu.SemaphoreType.DMA,
      ],
  )
  def scat_kernel(hs_hbm, src_hbm, dst_hbm, zeros_hbm, out_hbm,
                  src0, src1, dst0, dst1, idx0, idx1, rows0, rows1,
                  zbuf, acc, gs0, gs1, is0, is1):
    srcs = (src0, src1)
    dsts = (dst0, dst1)
    idxs = (idx0, idx1)
    rowsb = (rows0, rows1)
    gsems = (gs0, gs1)
    isems = (is0, is1)
    cid = lax.axis_index("c")
    sid = lax.axis_index("s")
    nbase = cid * hn
    base_r = sid * rows
    pltpu.sync_copy(zeros_hbm, zbuf)
    for z in range(rows // zr):
      pltpu.sync_copy(zbuf, acc.at[pl.ds(base_r + z * zr, zr)])
    @pl.when(sid == 0)
    def _():
      pltpu.sync_copy(zbuf.at[pl.ds(0, 8)], acc.at[pl.ds(hn, 8)])
    plsc.subcore_barrier()
    ebase = sid * ew

    def coff(c):
      return pl.multiple_of(ebase + c * k, 8)

    def remap(dstv, idxv):
      def rbody(j, carry2):
        dv = dstv[pl.ds(j * 16, 16)] - nbase
        ok = (dv >= 0) & (dv < hn)
        idxv[pl.ds(j * 16, 16)] = jnp.where(ok, dv, hn)
        return carry2

      lax.fori_loop(0, k // 16, rbody, 0)

    pltpu.sync_copy(src_hbm.at[pl.ds(coff(0), k)], src0)
    pltpu.sync_copy(dst_hbm.at[pl.ds(coff(0), k)], dst0)
    pltpu.async_copy(hs_hbm.at[src0], rows0, gs0)

    def body(g, carry):
      for b in (0, 1):
        c = 2 * g + b
        bn = 1 - b
        have_next = (c + 1 < nch) if b else True

        def prefetch_idx():
          off = coff(c + 1)
          pltpu.async_copy(src_hbm.at[pl.ds(off, k)], srcs[bn], isems[bn])
          pltpu.async_copy(dst_hbm.at[pl.ds(off, k)], dsts[bn], isems[bn])

        def launch_next_gather():
          off = coff(c + 1)
          pltpu.make_async_copy(
              src_hbm.at[pl.ds(off, k)], srcs[bn], isems[bn]).wait()
          pltpu.make_async_copy(
              dst_hbm.at[pl.ds(off, k)], dsts[bn], isems[bn]).wait()
          pltpu.async_copy(hs_hbm.at[srcs[bn]], rowsb[bn], gsems[bn])

        if have_next is True:
          prefetch_idx()
        else:
          pl.when(have_next)(prefetch_idx)
        remap(dsts[b], idxs[b])
        pltpu.make_async_copy(hs_hbm.at[srcs[b]], rowsb[b], gsems[b]).wait()
        if have_next is True:
          launch_next_gather()
        else:
          pl.when(have_next)(launch_next_gather)
        pltpu.sync_copy(rowsb[b], acc.at[idxs[b]], add=True)
      return carry

    lax.fori_loop(0, nch // 2, body, 0)
    plsc.subcore_barrier()
    for z in range(rows // zr):
      pltpu.sync_copy(acc.at[pl.ds(base_r + z * zr, zr)], zbuf)
      pltpu.sync_copy(zbuf, out_hbm.at[pl.ds(nbase + base_r + z * zr, zr)])

  return scat_kernel


def _tc_a(xp, d0, d1):
  n_pad, df = xp.shape

  def body(x_ref, d0_ref, d1_ref, xs_ref, dinv_ref):
    deg = d0_ref[...] + d1_ref[...] + 1.0
    dinv = lax.rsqrt(deg)
    xs_ref[...] = dinv * x_ref[...]
    dinv_ref[...] = dinv

  return pl.pallas_call(
      body,
      out_shape=[
          jax.ShapeDtypeStruct((n_pad, df), jnp.float32),
          jax.ShapeDtypeStruct((n_pad, 1), jnp.float32),
      ],
  )(xp, d0, d1)


def _tc_b(s1, xp, dinv, w1, b1, w2):
  n_pad, df = xp.shape

  def body(a_ref, x_ref, d_ref, w1_ref, b1_ref, w2_ref, hs_ref, self_ref):
    dv = d_ref[...]
    t = dv * a_ref[...] + (dv * dv) * x_ref[...]
    z1 = jnp.maximum(
        jnp.dot(t, w1_ref[...], preferred_element_type=jnp.float32)
        + b1_ref[...], 0.0)
    h2 = jnp.dot(z1, w2_ref[...], preferred_element_type=jnp.float32)
    hs_ref[...] = dv * h2
    self_ref[...] = (dv * dv) * h2

  return pl.pallas_call(
      body,
      out_shape=[
          jax.ShapeDtypeStruct((n_pad, df), jnp.float32),
          jax.ShapeDtypeStruct((n_pad, df), jnp.float32),
      ],
  )(s1, xp, dinv, w1, b1, w2)


def _tc_c(s2, self2, dinv, b2):
  n_pad, df = s2.shape

  def body(a_ref, s_ref, d_ref, bias_ref, out_ref):
    out_ref[...] = d_ref[...] * a_ref[...] + s_ref[...] + bias_ref[...]

  return pl.pallas_call(
      body,
      out_shape=jax.ShapeDtypeStruct((n_pad, df), jnp.float32),
  )(s2, self2, dinv, b2)


def kernel(x, pos_edge_index, W1, b1, W2, b2):
  n, df = x.shape
  dh = W1.shape[1]
  e = pos_edge_index.shape[1]
  n_pad = _np_pad(n)

  k_deg = 2048
  k128 = 256
  kmax = max(k_deg, k128)
  e_pad = -(-e // (NW * kmax)) * (NW * kmax)

  src = pos_edge_index[0]
  dst = pos_edge_index[1]
  if e_pad != e:
    pad = jnp.full((e_pad - e,), n, dtype=jnp.int32)
    src = jnp.concatenate([src, pad])
    dst = jnp.concatenate([dst, pad])

  xp = jnp.pad(x, ((0, n_pad - n), (0, 0)))
  rows = n_pad // NTILE
  z1d = jnp.zeros((rows,), jnp.float32)
  z128 = jnp.zeros((n_pad // 2 // NTILE // 2, df), jnp.float32)
  ones = jnp.ones((k_deg,), jnp.float32)

  degp0, degp1 = _make_deg(n_pad, e_pad, k_deg)(dst, ones, z1d)
  d0 = degp0[:, None]
  d1 = degp1[:, None]

  xs, dinv = _tc_a(xp, d0, d1)
  scat = _make_scatter(n_pad, e_pad, df, k128)
  s1 = scat(xs, src, dst, z128)
  hs2, self2 = _tc_b(s1, xp, dinv, W1, b1.reshape(1, dh), W2)
  s2 = scat(hs2, src, dst, z128)
  out = _tc_c(s2, self2, dinv, b2.reshape(1, df))
  return out[:n]


# speedup vs baseline: 5.8454x; 1.0028x over previous
import functools

import jax
import jax.numpy as jnp
from jax import lax
from jax.experimental import pallas as pl
from jax.experimental.pallas import tpu as pltpu
from jax.experimental.pallas import tpu_sc as plsc

NW = 32
NTILE = 16


def _np_pad(n):
  return (n // 512 + 1) * 512


def _make_deg(n_pad, e_pad, k):
  ew = e_pad // NW
  nch = ew // k
  rows = n_pad // NTILE
  mesh = plsc.VectorSubcoreMesh(core_axis_name="c", subcore_axis_name="s")

  @functools.partial(
      pl.kernel, mesh=mesh,
      out_type=[
          jax.ShapeDtypeStruct((n_pad,), jnp.float32),
          jax.ShapeDtypeStruct((n_pad,), jnp.float32),
      ],
      scratch_types=[
          pltpu.VMEM((k,), jnp.int32),
          pltpu.VMEM((k,), jnp.float32),
          pltpu.VMEM((n_pad // NTILE,), jnp.float32),
          pltpu.VMEM_SHARED((n_pad,), jnp.float32),
      ],
  )
  def deg_kernel(dst_hbm, ones_hbm, zeros_hbm, out0_hbm, out1_hbm,
                 dstv, onesv, zbuf, acc):
    cid = lax.axis_index("c")
    sid = lax.axis_index("s")
    wid = sid * 2 + cid
    base_r = sid * rows
    pltpu.sync_copy(zeros_hbm, zbuf)
    pltpu.sync_copy(zbuf, acc.at[pl.ds(base_r, rows)])
    pltpu.sync_copy(ones_hbm, onesv)
    plsc.subcore_barrier()
    ebase = wid * ew

    def body(c, carry):
      off = pl.multiple_of(ebase + c * k, 8)
      pltpu.sync_copy(dst_hbm.at[pl.ds(off, k)], dstv)
      pltpu.sync_copy(onesv, acc.at[dstv], add=True)
      return carry

    lax.fori_loop(0, nch, body, 0)
    plsc.subcore_barrier()
    pltpu.sync_copy(acc.at[pl.ds(base_r, rows)], zbuf)

    @pl.when(cid == 0)
    def _():
      pltpu.sync_copy(zbuf, out0_hbm.at[pl.ds(base_r, rows)])

    @pl.when(cid == 1)
    def _():
      pltpu.sync_copy(zbuf, out1_hbm.at[pl.ds(base_r, rows)])

  return deg_kernel


def _make_scatter(n_pad, e_pad, d, k):
  ew = e_pad // NTILE
  nch = ew // k
  assert nch % 2 == 0
  hn = n_pad // 2
  rows = hn // NTILE
  zr = rows // 2
  acc_rows = hn + 8
  mesh = plsc.VectorSubcoreMesh(core_axis_name="c", subcore_axis_name="s")

  @functools.partial(
      pl.kernel, mesh=mesh,
      out_type=jax.ShapeDtypeStruct((n_pad, d), jnp.float32),
      scratch_types=[
          pltpu.VMEM((k,), jnp.int32),
          pltpu.VMEM((k,), jnp.int32),
          pltpu.VMEM((k,), jnp.int32),
          pltpu.VMEM((k,), jnp.int32),
          pltpu.VMEM((k,), jnp.int32),
          pltpu.VMEM((k,), jnp.int32),
          pltpu.VMEM((k, d), jnp.float32),
          pltpu.VMEM((k, d), jnp.float32),
          pltpu.VMEM((zr, d), jnp.float32),
          pltpu.VMEM_SHARED((acc_rows, d), jnp.float32),
          pltpu.SemaphoreType.DMA,
          pltpu.SemaphoreType.DMA,
          pltpu.SemaphoreType.DMA,
          pltpu.SemaphoreType.DMA,
          pltpu.SemaphoreType.DMA,
          pltpu.SemaphoreType.DMA,
      ],
  )
  def scat_kernel(hs_hbm, src_hbm, dst_hbm, zeros_hbm, out_hbm,
                  src0, src1, dst0, dst1, idx0, idx1, rows0, rows1,
                  zbuf, acc, gs0, gs1, is0, is1, ss0, ss1):
    srcs = (src0, src1)
    dsts = (dst0, dst1)
    idxs = (idx0, idx1)
    rowsb = (rows0, rows1)
    gsems = (gs0, gs1)
    isems = (is0, is1)
    ssems = (ss0, ss1)
    cid = lax.axis_index("c")
    sid = lax.axis_index("s")
    nbase = cid * hn
    base_r = sid * rows
    pltpu.sync_copy(zeros_hbm, zbuf)
    for z in range(rows // zr):
      pltpu.sync_copy(zbuf, acc.at[pl.ds(base_r + z * zr, zr)])
    @pl.when(sid == 0)
    def _():
      pltpu.sync_copy(zbuf.at[pl.ds(0, 8)], acc.at[pl.ds(hn, 8)])
    plsc.subcore_barrier()
    ebase = sid * ew

    def coff(c):
      return pl.multiple_of(ebase + c * k, 8)

    uhn = jnp.uint32(hn)

    def remap(dstv, idxv):
      for j in range(k // 16):
        dv = dstv[pl.ds(j * 16, 16)] - nbase
        ok = plsc.bitcast(dv, jnp.uint32) < uhn
        idxv[pl.ds(j * 16, 16)] = jnp.where(ok, dv, hn)

    pltpu.sync_copy(src_hbm.at[pl.ds(coff(0), k)], src0)
    pltpu.sync_copy(dst_hbm.at[pl.ds(coff(0), k)], dst0)
    pltpu.async_copy(hs_hbm.at[src0], rows0, gs0)

    def body(g, carry):
      for b in (0, 1):
        c = 2 * g + b
        bn = 1 - b
        have_next = (c + 1 < nch) if b else True
        have_prev2 = (c >= 1) if b == 0 else True

        def prefetch_idx():
          off = coff(c + 1)
          pltpu.async_copy(src_hbm.at[pl.ds(off, k)], srcs[bn], isems[bn])
          pltpu.async_copy(dst_hbm.at[pl.ds(off, k)], dsts[bn], isems[bn])

        def launch_next_gather():
          off = coff(c + 1)
          pltpu.make_async_copy(
              src_hbm.at[pl.ds(off, k)], srcs[bn], isems[bn]).wait()
          pltpu.make_async_copy(
              dst_hbm.at[pl.ds(off, k)], dsts[bn], isems[bn]).wait()
          pltpu.async_copy(hs_hbm.at[srcs[bn]], rowsb[bn], gsems[bn])

        def drain_prev2_scatter():
          pltpu.make_async_copy(
              rowsb[bn], acc.at[idxs[bn]], ssems[bn]).wait()

        if have_next is True:
          prefetch_idx()
        else:
          pl.when(have_next)(prefetch_idx)
        remap(dsts[b], idxs[b])
        if have_prev2 is True:
          drain_prev2_scatter()
        else:
          pl.when(have_prev2)(drain_prev2_scatter)
        if have_next is True:
          launch_next_gather()
        else:
          pl.when(have_next)(launch_next_gather)
        pltpu.make_async_copy(hs_hbm.at[srcs[b]], rowsb[b], gsems[b]).wait()
        pltpu.async_copy(rowsb[b], acc.at[idxs[b]], ssems[b], add=True)
      return carry

    lax.fori_loop(0, nch // 2, body, 0)
    pltpu.make_async_copy(rowsb[1], acc.at[idxs[1]], ssems[1]).wait()
    plsc.subcore_barrier()
    for z in range(rows // zr):
      pltpu.sync_copy(acc.at[pl.ds(base_r + z * zr, zr)], zbuf)
      pltpu.sync_copy(zbuf, out_hbm.at[pl.ds(nbase + base_r + z * zr, zr)])

  return scat_kernel


def _tc_a(xp, d0, d1):
  n_pad, df = xp.shape

  def body(x_ref, d0_ref, d1_ref, xs_ref, dinv_ref):
    deg = d0_ref[...] + d1_ref[...] + 1.0
    dinv = lax.rsqrt(deg)
    xs_ref[...] = dinv * x_ref[...]
    dinv_ref[...] = dinv

  return pl.pallas_call(
      body,
      out_shape=[
          jax.ShapeDtypeStruct((n_pad, df), jnp.float32),
          jax.ShapeDtypeStruct((n_pad, 1), jnp.float32),
      ],
  )(xp, d0, d1)


def _tc_b(s1, xp, dinv, w1, b1, w2):
  n_pad, df = xp.shape

  def body(a_ref, x_ref, d_ref, w1_ref, b1_ref, w2_ref, hs_ref, self_ref):
    dv = d_ref[...]
    t = dv * a_ref[...] + (dv * dv) * x_ref[...]
    z1 = jnp.maximum(
        jnp.dot(t, w1_ref[...], preferred_element_type=jnp.float32)
        + b1_ref[...], 0.0)
    h2 = jnp.dot(z1, w2_ref[...], preferred_element_type=jnp.float32)
    hs_ref[...] = dv * h2
    self_ref[...] = (dv * dv) * h2

  return pl.pallas_call(
      body,
      out_shape=[
          jax.ShapeDtypeStruct((n_pad, df), jnp.float32),
          jax.ShapeDtypeStruct((n_pad, df), jnp.float32),
      ],
  )(s1, xp, dinv, w1, b1, w2)


def _tc_c(s2, self2, dinv, b2):
  n_pad, df = s2.shape

  def body(a_ref, s_ref, d_ref, bias_ref, out_ref):
    out_ref[...] = d_ref[...] * a_ref[...] + s_ref[...] + bias_ref[...]

  return pl.pallas_call(
      body,
      out_shape=jax.ShapeDtypeStruct((n_pad, df), jnp.float32),
  )(s2, self2, dinv, b2)


def kernel(x, pos_edge_index, W1, b1, W2, b2):
  n, df = x.shape
  dh = W1.shape[1]
  e = pos_edge_index.shape[1]
  n_pad = _np_pad(n)

  k_deg = 2048
  k128 = 256
  kmax = max(k_deg, k128)
  e_pad = -(-e // (NW * kmax)) * (NW * kmax)

  src = pos_edge_index[0]
  dst = pos_edge_index[1]
  if e_pad != e:
    pad = jnp.full((e_pad - e,), n, dtype=jnp.int32)
    src = jnp.concatenate([src, pad])
    dst = jnp.concatenate([dst, pad])

  xp = jnp.pad(x, ((0, n_pad - n), (0, 0)))
  rows = n_pad // NTILE
  z1d = jnp.zeros((rows,), jnp.float32)
  z128 = jnp.zeros((n_pad // 2 // NTILE // 2, df), jnp.float32)
  ones = jnp.ones((k_deg,), jnp.float32)

  degp0, degp1 = _make_deg(n_pad, e_pad, k_deg)(dst, ones, z1d)
  d0 = degp0[:, None]
  d1 = degp1[:, None]

  xs, dinv = _tc_a(xp, d0, d1)
  scat = _make_scatter(n_pad, e_pad, df, k128)
  s1 = scat(xs, src, dst, z128)
  hs2, self2 = _tc_b(s1, xp, dinv, W1, b1.reshape(1, dh), W2)
  s2 = scat(hs2, src, dst, z128)
  out = _tc_c(s2, self2, dinv, b2.reshape(1, df))
  return out[:n]


# speedup vs baseline: 7.2087x; 1.2332x over previous
import functools

import jax
import jax.numpy as jnp
from jax import lax
from jax.experimental import pallas as pl
from jax.experimental.pallas import tpu as pltpu
from jax.experimental.pallas import tpu_sc as plsc

NW = 32
NTILE = 16


def _np_pad(n):
  return (n // 512 + 1) * 512


def _make_deg(n_pad, e_pad, k):
  ew = e_pad // NW
  nch = ew // k
  rows = n_pad // NTILE
  mesh = plsc.VectorSubcoreMesh(core_axis_name="c", subcore_axis_name="s")

  @functools.partial(
      pl.kernel, mesh=mesh,
      out_type=[
          jax.ShapeDtypeStruct((n_pad,), jnp.float32),
          jax.ShapeDtypeStruct((n_pad,), jnp.float32),
      ],
      scratch_types=[
          pltpu.VMEM((k,), jnp.int32),
          pltpu.VMEM((k,), jnp.float32),
          pltpu.VMEM((n_pad // NTILE,), jnp.float32),
          pltpu.VMEM_SHARED((n_pad,), jnp.float32),
      ],
  )
  def deg_kernel(dst_hbm, ones_hbm, zeros_hbm, out0_hbm, out1_hbm,
                 dstv, onesv, zbuf, acc):
    cid = lax.axis_index("c")
    sid = lax.axis_index("s")
    wid = sid * 2 + cid
    base_r = sid * rows
    pltpu.sync_copy(zeros_hbm, zbuf)
    pltpu.sync_copy(zbuf, acc.at[pl.ds(base_r, rows)])
    pltpu.sync_copy(ones_hbm, onesv)
    plsc.subcore_barrier()
    ebase = wid * ew

    def body(c, carry):
      off = pl.multiple_of(ebase + c * k, 8)
      pltpu.sync_copy(dst_hbm.at[pl.ds(off, k)], dstv)
      pltpu.sync_copy(onesv, acc.at[dstv], add=True)
      return carry

    lax.fori_loop(0, nch, body, 0)
    plsc.subcore_barrier()
    pltpu.sync_copy(acc.at[pl.ds(base_r, rows)], zbuf)

    @pl.when(cid == 0)
    def _():
      pltpu.sync_copy(zbuf, out0_hbm.at[pl.ds(base_r, rows)])

    @pl.when(cid == 1)
    def _():
      pltpu.sync_copy(zbuf, out1_hbm.at[pl.ds(base_r, rows)])

  return deg_kernel


def _make_partition(n_pad, e_pad, ks):
  cap = e_pad // NW
  capp = cap + ks
  kp = 2048
  hn = n_pad // 2
  n_trash = hn
  mesh = plsc.VectorSubcoreMesh(core_axis_name="c", subcore_axis_name="s")

  sp_words = NTILE * 2 * capp
  dead = sp_words
  dq = capp // 4

  @functools.partial(
      pl.kernel, mesh=mesh,
      out_type=[
          jax.ShapeDtypeStruct((4 * NTILE * capp,), jnp.int32),
          jax.ShapeDtypeStruct((NW * 16,), jnp.int32),
      ],
      scratch_types=[
          pltpu.VMEM((kp,), jnp.int32),
          pltpu.VMEM((kp,), jnp.int32),
          pltpu.VMEM((kp,), jnp.int32),
          pltpu.VMEM((kp,), jnp.int32),
          pltpu.VMEM((kp,), jnp.int32),
          pltpu.VMEM((dq,), jnp.int32),
          pltpu.VMEM((16,), jnp.int32),
          pltpu.VMEM_SHARED((sp_words + 8,), jnp.int32),
      ],
  )
  def part_kernel(src_hbm, dst_hbm, pk_out, cnt_out,
                  sv, dv, pvlo, pvhi, pkv, dmp, cbuf, sp):
    cid = lax.axis_index("c")
    sid = lax.axis_index("s")
    wid = sid * 2 + cid
    ebase = wid * cap
    lanes = lax.broadcasted_iota(jnp.int32, (16,), 0)
    lanes15 = jnp.full((16,), 15, jnp.int32)
    lo_base = sid * (2 * capp)
    hi_base = lo_base + capp

    def chunk(ci, carry):
      off = pl.multiple_of(ebase + ci * kp, 8)
      pltpu.sync_copy(src_hbm.at[pl.ds(off, kp)], sv)
      pltpu.sync_copy(dst_hbm.at[pl.ds(off, kp)], dv)

      def vec(j, carry2):
        olo, ohi = carry2
        sl = pl.ds(j * 16, 16)
        s16 = sv[sl]
        d16 = dv[sl]
        lo = d16 < hn
        s = jnp.where(lo, 1, 0)
        for sft in (1, 2, 4, 8):
          sh = s.at[jnp.maximum(lanes - sft, 0)].get(
              mode="promise_in_bounds")
          s = s + jnp.where(lanes >= sft, sh, 0)
        nlo = s.at[lanes15].get(mode="promise_in_bounds")
        pvlo[sl] = jnp.where(lo, lo_base + olo + s - 1, dead)
        pvhi[sl] = jnp.where(lo, dead, hi_base + ohi + lanes - s)
        pkv[sl] = s16 * 8192 + jnp.where(lo, d16, d16 - hn)
        return (olo + nlo, ohi + (16 - nlo))

      carry = lax.fori_loop(0, kp // 16, vec, carry)
      pltpu.sync_copy(pkv, sp.at[pvlo])
      pltpu.sync_copy(pkv, sp.at[pvhi])
      return carry

    zero16 = jnp.zeros((16,), jnp.int32)
    olo, ohi = lax.fori_loop(0, cap // kp, chunk, (zero16, zero16))

    trash = (n_pad - 1) * 8192 + n_trash + (lanes & 7)

    def padv(t, carry2):
      sl = pl.ds(t * 16, 16)
      tt = lanes + t * 16
      in_pad = t < (ks // 16)
      pvlo[sl] = jnp.where(in_pad, lo_base + olo + tt, dead)
      pvhi[sl] = jnp.where(in_pad, hi_base + ohi + tt, dead)
      pkv[sl] = trash
      return carry2

    lax.fori_loop(0, kp // 16, padv, 0)
    pltpu.sync_copy(pkv, sp.at[pvlo])
    pltpu.sync_copy(pkv, sp.at[pvhi])

    cbuf[pl.ds(0, 16)] = jnp.where(
        lanes == 0, olo, jnp.where(lanes == 1, ohi, 0))
    pltpu.sync_copy(cbuf, cnt_out.at[pl.ds(wid * 16, 16)])

    for half in (0, 1):
      hbase = ((half * 2 + cid) * NTILE + sid) * capp
      for q in range(4):
        pltpu.sync_copy(
            sp.at[pl.ds(lo_base + half * capp + q * dq, dq)], dmp)
        pltpu.sync_copy(dmp, pk_out.at[pl.ds(hbase + q * dq, dq)])

  return part_kernel


def _make_scatter(n_pad, e_pad, d, k):
  cap = e_pad // NW
  capp = cap + k
  hn = n_pad // 2
  rows = hn // NTILE
  zr = rows // 2
  acc_rows = hn + 8
  mesh = plsc.VectorSubcoreMesh(core_axis_name="c", subcore_axis_name="s")

  @functools.partial(
      pl.kernel, mesh=mesh,
      out_type=jax.ShapeDtypeStruct((n_pad, d), jnp.float32),
      scratch_types=[
          pltpu.VMEM((k,), jnp.int32),
          pltpu.VMEM((k,), jnp.int32),
          pltpu.VMEM((k,), jnp.int32),
          pltpu.VMEM((k,), jnp.int32),
          pltpu.VMEM((k,), jnp.int32),
          pltpu.VMEM((k,), jnp.int32),
          pltpu.VMEM((16,), jnp.int32),
          pltpu.VMEM((k, d), jnp.float32),
          pltpu.VMEM((k, d), jnp.float32),
          pltpu.VMEM((zr, d), jnp.float32),
          pltpu.VMEM_SHARED((acc_rows, d), jnp.float32),
          pltpu.SemaphoreType.DMA,
          pltpu.SemaphoreType.DMA,
          pltpu.SemaphoreType.DMA,
          pltpu.SemaphoreType.DMA,
      ],
  )
  def scat_kernel(hs_hbm, pk_hbm, cnt_hbm, zeros_hbm, out_hbm,
                  pb0, pb1, src0, src1, dst0, dst1, cbuf, rows0, rows1,
                  zbuf, acc, gs0, gs1, ss0, ss1):
    pbs = (pb0, pb1)
    srcs = (src0, src1)
    dsts = (dst0, dst1)
    rowsb = (rows0, rows1)
    gsems = (gs0, gs1)
    ssems = (ss0, ss1)
    cid = lax.axis_index("c")
    sid = lax.axis_index("s")
    base_r = sid * rows
    lanes = lax.broadcasted_iota(jnp.int32, (16,), 0)
    pltpu.sync_copy(zeros_hbm, zbuf)
    for z in range(rows // zr):
      pltpu.sync_copy(zbuf, acc.at[pl.ds(base_r + z * zr, zr)])
    @pl.when(sid == 0)
    def _():
      pltpu.sync_copy(zbuf.at[pl.ds(0, 8)], acc.at[pl.ds(hn, 8)])
    plsc.subcore_barrier()

    sr13 = jnp.int32(13)
    m13 = jnp.int32(8191)

    def unpack(b):
      for j in range(k // 16):
        sl = pl.ds(j * 16, 16)
        p16 = pbs[b][sl]
        srcs[b][sl] = lax.shift_right_logical(p16, sr13)
        dsts[b][sl] = p16 & m13

    for li in (0, 1):
      lbase = ((cid * 2 + li) * NTILE + sid) * capp
      wid_src = sid * 2 + li
      pltpu.sync_copy(cnt_hbm.at[pl.ds(wid_src * 16, 16)], cbuf)
      cvec = cbuf[pl.ds(0, 16)]
      cnt = jnp.where(cid == 0, cvec[0], cvec[1])
      trip = jnp.maximum((cnt + k - 1) // k, 1)

      def coff(c):
        return pl.multiple_of(lbase + c * k, 8)

      pltpu.sync_copy(pk_hbm.at[pl.ds(coff(0), k)], pbs[0])
      unpack(0)
      pltpu.async_copy(hs_hbm.at[srcs[0]], rowsb[0], gsems[0])

      def wbody(c, trip_count):
        active = c < trip

        def run(b):
          bn = 1 - b
          have_next = (c + 1) < trip

          def fetch_next():
            @pl.when(c >= 1)
            def _():
              pltpu.make_async_copy(
                  rowsb[bn], acc.at[dsts[bn]], ssems[bn]).wait()

            off = coff(c + 1)
            pltpu.sync_copy(pk_hbm.at[pl.ds(off, k)], pbs[bn])
            unpack(bn)
            pltpu.async_copy(hs_hbm.at[srcs[bn]], rowsb[bn], gsems[bn])

          pl.when(have_next)(fetch_next)
          pltpu.make_async_copy(hs_hbm.at[srcs[b]], rowsb[b], gsems[b]).wait()
          pltpu.async_copy(rowsb[b], acc.at[dsts[b]], ssems[b], add=True)

        def run_active():
          pl.when(c % 2 == 0)(lambda: run(0))
          pl.when(c % 2 == 1)(lambda: run(1))

        pl.when(active)(run_active)
        return trip_count

      lax.fori_loop(0, capp // k, wbody, 0)

      def drain(b):
        pltpu.make_async_copy(rowsb[b], acc.at[dsts[b]], ssems[b]).wait()

      pl.when((trip >= 2) & (trip % 2 == 0))(lambda: drain(0))
      pl.when((trip >= 2) & (trip % 2 == 1))(lambda: drain(1))
      pl.when((trip - 1) % 2 == 0)(lambda: drain(0))
      pl.when((trip - 1) % 2 == 1)(lambda: drain(1))

    plsc.subcore_barrier()
    for z in range(rows // zr):
      pltpu.sync_copy(acc.at[pl.ds(base_r + z * zr, zr)], zbuf)
      pltpu.sync_copy(zbuf, out_hbm.at[pl.ds(cid * hn + base_r + z * zr, zr)])

  return scat_kernel


def _tc_a(xp, d0, d1):
  n_pad, df = xp.shape

  def body(x_ref, d0_ref, d1_ref, xs_ref, dinv_ref):
    deg = d0_ref[...] + d1_ref[...] + 1.0
    dinv = lax.rsqrt(deg)
    xs_ref[...] = dinv * x_ref[...]
    dinv_ref[...] = dinv

  return pl.pallas_call(
      body,
      out_shape=[
          jax.ShapeDtypeStruct((n_pad, df), jnp.float32),
          jax.ShapeDtypeStruct((n_pad, 1), jnp.float32),
      ],
  )(xp, d0, d1)


def _tc_b(s1, xp, dinv, w1, b1, w2):
  n_pad, df = xp.shape

  def body(a_ref, x_ref, d_ref, w1_ref, b1_ref, w2_ref, hs_ref, self_ref):
    dv = d_ref[...]
    t = dv * a_ref[...] + (dv * dv) * x_ref[...]
    z1 = jnp.maximum(
        jnp.dot(t, w1_ref[...], preferred_element_type=jnp.float32)
        + b1_ref[...], 0.0)
    h2 = jnp.dot(z1, w2_ref[...], preferred_element_type=jnp.float32)
    hs_ref[...] = dv * h2
    self_ref[...] = (dv * dv) * h2

  return pl.pallas_call(
      body,
      out_shape=[
          jax.ShapeDtypeStruct((n_pad, df), jnp.float32),
          jax.ShapeDtypeStruct((n_pad, df), jnp.float32),
      ],
  )(s1, xp, dinv, w1, b1, w2)


def _tc_c(s2, self2, dinv, b2):
  n_pad, df = s2.shape

  def body(a_ref, s_ref, d_ref, bias_ref, out_ref):
    out_ref[...] = d_ref[...] * a_ref[...] + s_ref[...] + bias_ref[...]

  return pl.pallas_call(
      body,
      out_shape=jax.ShapeDtypeStruct((n_pad, df), jnp.float32),
  )(s2, self2, dinv, b2)


def kernel(x, pos_edge_index, W1, b1, W2, b2):
  n, df = x.shape
  dh = W1.shape[1]
  e = pos_edge_index.shape[1]
  n_pad = _np_pad(n)

  k_deg = 2048
  k128 = 256
  kmax = max(k_deg, k128)
  e_pad = -(-e // (NW * kmax)) * (NW * kmax)

  src = pos_edge_index[0]
  dst = pos_edge_index[1]
  if e_pad != e:
    pad = jnp.full((e_pad - e,), n, dtype=jnp.int32)
    src = jnp.concatenate([src, pad])
    dst = jnp.concatenate([dst, pad])

  xp = jnp.pad(x, ((0, n_pad - n), (0, 0)))
  rows = n_pad // NTILE
  z1d = jnp.zeros((rows,), jnp.float32)
  z128 = jnp.zeros((n_pad // 2 // NTILE // 2, df), jnp.float32)
  ones = jnp.ones((k_deg,), jnp.float32)

  degp0, degp1 = _make_deg(n_pad, e_pad, k_deg)(dst, ones, z1d)
  d0 = degp0[:, None]
  d1 = degp1[:, None]

  pk, cnts = _make_partition(n_pad, e_pad, k128)(src, dst)
  xs, dinv = _tc_a(xp, d0, d1)
  scat = _make_scatter(n_pad, e_pad, df, k128)
  s1 = scat(xs, pk, cnts, z128)
  hs2, self2 = _tc_b(s1, xp, dinv, W1, b1.reshape(1, dh), W2)
  s2 = scat(hs2, pk, cnts, z128)
  out = _tc_c(s2, self2, dinv, b2.reshape(1, df))
  return out[:n]
